# baseline (speedup 1.0000x reference)
import functools

import jax
import jax.numpy as jnp
from jax import lax
from jax.experimental import pallas as pl
from jax.experimental.pallas import tpu as pltpu
from jax.experimental.pallas import tpu_sc as plsc

_NW = 32
_B = 128
_LANES = 16


def _pad_shape(n):
    npad = -(-(n + 1) // (_LANES * 8)) * _LANES * 8
    npad = -(-npad // 128) * 128
    return npad, npad // _LANES


def _sc_mesh():
    return plsc.VectorSubcoreMesh(core_axis_name="c", subcore_axis_name="s")


def _sc_degree(c3, n):
    nw, ch, b = c3.shape
    npad, per_sub = _pad_shape(n)

    @functools.partial(
        pl.kernel,
        mesh=_sc_mesh(),
        out_type=jax.ShapeDtypeStruct((2, npad), jnp.float32),
        scratch_types=[
            pltpu.VMEM((ch, b), jnp.int32),
            pltpu.VMEM((b,), jnp.float32),
            pltpu.VMEM_SHARED((npad,), jnp.float32),
        ],
    )
    def deg_kernel(c_hbm, deg_hbm, cidx, ones, acc):
        cid = lax.axis_index("c")
        sid = lax.axis_index("s")
        wid = sid * 2 + cid

        def fill(val):
            def step(i, _):
                ones[pl.ds(i * _LANES, _LANES)] = jnp.full(
                    (_LANES,), val, jnp.float32)
                return 0
            lax.fori_loop(0, b // _LANES, step, 0)

        fill(0.0)

        def zero(i, _):
            pltpu.sync_copy(ones, acc.at[pl.ds(sid * per_sub + i * b, b)])
            return 0

        lax.fori_loop(0, per_sub // b, zero, 0)
        rem = per_sub % b
        if rem:
            pltpu.sync_copy(
                ones.at[pl.ds(0, rem)],
                acc.at[pl.ds(sid * per_sub + (per_sub // b) * b, rem)])
        fill(1.0)
        plsc.subcore_barrier()
        pltpu.sync_copy(c_hbm.at[wid], cidx)

        def chunk(j, _):
            pltpu.sync_copy(ones, acc.at[cidx.at[j]], add=True)
            return 0

        lax.fori_loop(0, ch, chunk, 0)
        plsc.subcore_barrier()

        @pl.when(sid == 0)
        def _():
            pltpu.sync_copy(acc, deg_hbm.at[cid])

    return deg_kernel(c3)


def _sc_aggregate(y, rflat, cflat, ch):
    n, d = y.shape
    b = _B
    npad, per_sub = _pad_shape(n)
    assert ch % 2 == 0 and ch >= 4

    @functools.partial(
        pl.kernel,
        mesh=_sc_mesh(),
        out_type=jax.ShapeDtypeStruct((2, npad, d), jnp.float32),
        scratch_types=[
            pltpu.VMEM((b,), jnp.int32),
            pltpu.VMEM((b,), jnp.int32),
            pltpu.VMEM((b,), jnp.int32),
            pltpu.VMEM((b,), jnp.int32),
            pltpu.VMEM((b, d), jnp.float32),
            pltpu.VMEM((b, d), jnp.float32),
            pltpu.VMEM_SHARED((npad, d), jnp.float32),
            pltpu.SemaphoreType.DMA,
            pltpu.SemaphoreType.DMA,
            pltpu.SemaphoreType.DMA,
            pltpu.SemaphoreType.DMA,
            pltpu.SemaphoreType.DMA,
            pltpu.SemaphoreType.DMA,
        ],
    )
    def agg_kernel(y_hbm, r_hbm, c_hbm, out_hbm,
                   rb0, rb1, cb0, cb1, rows0, rows1, acc,
                   sir0, sir1, sic0, sic1, sg0, sg1):
        cid = lax.axis_index("c")
        sid = lax.axis_index("s")
        wid = sid * 2 + cid
        base = wid * ch * b

        def zrow(i, _):
            for k in range(d // _LANES):
                rows0[i, pl.ds(k * _LANES, _LANES)] = jnp.zeros(
                    (_LANES,), jnp.float32)
            return 0

        lax.fori_loop(0, b, zrow, 0)

        def zacc(i, _):
            pltpu.sync_copy(rows0, acc.at[pl.ds(sid * per_sub + i * b, b)])
            return 0

        lax.fori_loop(0, per_sub // b, zacc, 0)
        zrem = per_sub % b
        if zrem:
            pltpu.sync_copy(
                rows0.at[pl.ds(0, zrem)],
                acc.at[pl.ds(sid * per_sub + (per_sub // b) * b, zrem)])
        plsc.subcore_barrier()

        def load_r(j, buf, sem):
            pltpu.async_copy(r_hbm.at[pl.ds(base + j * b, b)], buf, sem)

        def load_c(j, buf, sem):
            pltpu.async_copy(c_hbm.at[pl.ds(base + j * b, b)], buf, sem)

        def wait(hbm, buf, sem):
            pltpu.make_async_copy(hbm.at[pl.ds(base, b)], buf, sem).wait()

        def gather(buf, rows, sem):
            pltpu.async_copy(y_hbm.at[buf], rows, sem)

        def wait_gather(buf, rows, sem):
            pltpu.make_async_copy(y_hbm.at[buf], rows, sem).wait()

        def scatter(rows, buf):
            pltpu.sync_copy(rows, acc.at[buf], add=True)

        load_r(0, rb0, sir0)
        load_c(0, cb0, sic0)
        load_r(1, rb1, sir1)
        load_c(1, cb1, sic1)
        wait(r_hbm, rb0, sir0)
        gather(rb0, rows0, sg0)

        def pair(i, _):
            j = 2 * i
            wait_gather(rb0, rows0, sg0)
            load_r(j + 2, rb0, sir0)
            wait(c_hbm, cb0, sic0)
            scatter(rows0, cb0)
            load_c(j + 2, cb0, sic0)
            wait(r_hbm, rb1, sir1)
            gather(rb1, rows1, sg1)
            wait_gather(rb1, rows1, sg1)
            load_r(j + 3, rb1, sir1)
            wait(c_hbm, cb1, sic1)
            scatter(rows1, cb1)
            load_c(j + 3, cb1, sic1)
            wait(r_hbm, rb0, sir0)
            gather(rb0, rows0, sg0)
            return 0

        lax.fori_loop(0, ch // 2 - 1, pair, 0)
        wait(r_hbm, rb1, sir1)
        gather(rb1, rows1, sg1)
        wait_gather(rb0, rows0, sg0)
        wait(c_hbm, cb0, sic0)
        scatter(rows0, cb0)
        wait_gather(rb1, rows1, sg1)
        wait(c_hbm, cb1, sic1)
        scatter(rows1, cb1)
        plsc.subcore_barrier()
        pltpu.sync_copy(acc.at[pl.ds(sid * per_sub, per_sub)],
                        out_hbm.at[cid, pl.ds(sid * per_sub, per_sub)])

    return agg_kernel(y, rflat, cflat)


def _tc_pre(x, degp3, w0t, fcwt, fcb2, blk=1000):
    n, d = x.shape
    g = n // blk

    def body(x_ref, degp_ref, w0t_ref, fcwt_ref, fcb_ref,
             y0_ref, res_ref, dinv_ref):
        deg = degp_ref[0] + degp_ref[1] + 1.0
        dinv = lax.rsqrt(deg)
        dinv_ref[...] = dinv
        xb = x_ref[...]
        y0_ref[...] = jnp.dot(xb, w0t_ref[...],
                              preferred_element_type=jnp.float32) * dinv
        res_ref[...] = jnp.dot(xb, fcwt_ref[...],
                               preferred_element_type=jnp.float32) + fcb_ref[...]

    return pl.pallas_call(
        body,
        grid=(g,),
        in_specs=[
            pl.BlockSpec((blk, d), lambda i: (i, 0)),
            pl.BlockSpec((2, blk, 1), lambda i: (0, i, 0)),
            pl.BlockSpec((d, d), lambda i: (0, 0)),
            pl.BlockSpec((d, d), lambda i: (0, 0)),
            pl.BlockSpec((1, d), lambda i: (0, 0)),
        ],
        out_specs=[
            pl.BlockSpec((blk, d), lambda i: (i, 0)),
            pl.BlockSpec((blk, d), lambda i: (i, 0)),
            pl.BlockSpec((blk, 1), lambda i: (i, 0)),
        ],
        out_shape=[
            jax.ShapeDtypeStruct((n, d), jnp.float32),
            jax.ShapeDtypeStruct((n, d), jnp.float32),
            jax.ShapeDtypeStruct((n, 1), jnp.float32),
        ],
    )(x, degp3, w0t, fcwt, fcb2)


def _tc_mid(aggp, y0, dinv, b02, w1t, blk=1000):
    n, d = y0.shape

    def body(aggp_ref, y0_ref, dinv_ref, b0_ref, w1t_ref, y1_ref):
        dinv = dinv_ref[...]
        h = (aggp_ref[0] + aggp_ref[1] + y0_ref[...]) * dinv + b0_ref[...]
        h = jnp.maximum(h, 0.0)
        y1_ref[...] = jnp.dot(h, w1t_ref[...],
                              preferred_element_type=jnp.float32) * dinv

    return pl.pallas_call(
        body,
        grid=(n // blk,),
        in_specs=[
            pl.BlockSpec((2, blk, d), lambda i: (0, i, 0)),
            pl.BlockSpec((blk, d), lambda i: (i, 0)),
            pl.BlockSpec((blk, 1), lambda i: (i, 0)),
            pl.BlockSpec((1, d), lambda i: (0, 0)),
            pl.BlockSpec((d, d), lambda i: (0, 0)),
        ],
        out_specs=pl.BlockSpec((blk, d), lambda i: (i, 0)),
        out_shape=jax.ShapeDtypeStruct((n, d), jnp.float32),
    )(aggp, y0, dinv, b02, w1t)


def _tc_post(aggp, y1, dinv, b12, res, blk=1000):
    n, d = y1.shape

    def body(aggp_ref, y1_ref, dinv_ref, b1_ref, res_ref, out_ref):
        h = (aggp_ref[0] + aggp_ref[1] + y1_ref[...]) * dinv_ref[...] \
            + b1_ref[...]
        out_ref[...] = jnp.maximum(h, 0.0) + res_ref[...]

    return pl.pallas_call(
        body,
        grid=(n // blk,),
        in_specs=[
            pl.BlockSpec((2, blk, d), lambda i: (0, i, 0)),
            pl.BlockSpec((blk, d), lambda i: (i, 0)),
            pl.BlockSpec((blk, 1), lambda i: (i, 0)),
            pl.BlockSpec((1, d), lambda i: (0, 0)),
            pl.BlockSpec((blk, d), lambda i: (i, 0)),
        ],
        out_specs=pl.BlockSpec((blk, d), lambda i: (i, 0)),
        out_shape=jax.ShapeDtypeStruct((n, d), jnp.float32),
    )(aggp, y1, dinv, b12, res)


def kernel(x, edge_index, W0, b0, W1, b1, fcW, fcb):
    n, d = x.shape
    e = edge_index.shape[1]
    ch = -(-e // (_NW * _B))
    ch += ch % 2
    ep = _NW * ch * _B
    r = edge_index[0].astype(jnp.int32)
    c = edge_index[1].astype(jnp.int32)
    pad = ep - e
    rflat = jnp.concatenate([r, jnp.zeros((pad,), jnp.int32)])
    cflat = jnp.concatenate([c, jnp.full((pad,), n, jnp.int32)])
    c3 = cflat.reshape(_NW, ch, _B)

    degp = _sc_degree(c3, n)
    degp3 = degp[:, :n].reshape(2, n, 1)
    y0, res, dinv = _tc_pre(x, degp3, W0.T, fcW.T, fcb.reshape(1, d))
    agg0 = _sc_aggregate(y0, rflat, cflat, ch)[:, :n]
    y1 = _tc_mid(agg0, y0, dinv, b0.reshape(1, d), W1.T)
    agg1 = _sc_aggregate(y1, rflat, cflat, ch)[:, :n]
    return _tc_post(agg1, y1, dinv, b1.reshape(1, d), res)


# speedup vs baseline: 1.1326x; 1.1326x over previous
import functools

import jax
import jax.numpy as jnp
from jax import lax
from jax.experimental import pallas as pl
from jax.experimental.pallas import tpu as pltpu
from jax.experimental.pallas import tpu_sc as plsc

_NW = 32
_B = 128
_LANES = 16


def _pad_shape(n):
    npad = -(-(n + 1) // (_LANES * 8)) * _LANES * 8
    npad = -(-npad // 128) * 128
    return npad, npad // _LANES


def _sc_mesh():
    return plsc.VectorSubcoreMesh(core_axis_name="c", subcore_axis_name="s")


def _sc_degree(c3, n):
    nw, ch, b = c3.shape
    npad, per_sub = _pad_shape(n)

    @functools.partial(
        pl.kernel,
        mesh=_sc_mesh(),
        out_type=jax.ShapeDtypeStruct((2, npad), jnp.float32),
        scratch_types=[
            pltpu.VMEM((ch, b), jnp.int32),
            pltpu.VMEM((b,), jnp.float32),
            pltpu.VMEM_SHARED((npad,), jnp.float32),
        ],
    )
    def deg_kernel(c_hbm, deg_hbm, cidx, ones, acc):
        cid = lax.axis_index("c")
        sid = lax.axis_index("s")
        wid = sid * 2 + cid

        def fill(val):
            def step(i, _):
                ones[pl.ds(i * _LANES, _LANES)] = jnp.full(
                    (_LANES,), val, jnp.float32)
                return 0
            lax.fori_loop(0, b // _LANES, step, 0)

        fill(0.0)

        def zero(i, _):
            pltpu.sync_copy(ones, acc.at[pl.ds(sid * per_sub + i * b, b)])
            return 0

        lax.fori_loop(0, per_sub // b, zero, 0)
        rem = per_sub % b
        if rem:
            pltpu.sync_copy(
                ones.at[pl.ds(0, rem)],
                acc.at[pl.ds(sid * per_sub + (per_sub // b) * b, rem)])
        fill(1.0)
        plsc.subcore_barrier()
        pltpu.sync_copy(c_hbm.at[wid], cidx)

        def chunk(j, _):
            pltpu.sync_copy(ones, acc.at[cidx.at[j]], add=True)
            return 0

        lax.fori_loop(0, ch, chunk, 0)
        plsc.subcore_barrier()

        @pl.when(sid == 0)
        def _():
            pltpu.sync_copy(acc, deg_hbm.at[cid])

    return deg_kernel(c3)


def _sc_aggregate(y, rflat, cflat, ch):
    n, d = y.shape
    b = _B
    npad, per_sub = _pad_shape(n)
    assert ch % 2 == 0 and ch >= 4

    @functools.partial(
        pl.kernel,
        mesh=_sc_mesh(),
        out_type=jax.ShapeDtypeStruct((2, npad, d), jnp.float32),
        scratch_types=[
            pltpu.VMEM((b,), jnp.int32),
            pltpu.VMEM((b,), jnp.int32),
            pltpu.VMEM((b,), jnp.int32),
            pltpu.VMEM((b,), jnp.int32),
            pltpu.VMEM((b, d), jnp.float32),
            pltpu.VMEM((b, d), jnp.float32),
            pltpu.VMEM_SHARED((npad, d), jnp.float32),
            pltpu.SemaphoreType.DMA,
            pltpu.SemaphoreType.DMA,
            pltpu.SemaphoreType.DMA,
            pltpu.SemaphoreType.DMA,
            pltpu.SemaphoreType.DMA,
            pltpu.SemaphoreType.DMA,
        ],
    )
    def agg_kernel(y_hbm, r_hbm, c_hbm, out_hbm,
                   rb0, rb1, cb0, cb1, rows0, rows1, acc,
                   sir0, sir1, sic0, sic1, sg0, sg1):
        cid = lax.axis_index("c")
        sid = lax.axis_index("s")
        wid = sid * 2 + cid
        base = wid * ch * b

        def zrow(i, _):
            for k in range(d // _LANES):
                rows0[i, pl.ds(k * _LANES, _LANES)] = jnp.zeros(
                    (_LANES,), jnp.float32)
            return 0

        lax.fori_loop(0, b, zrow, 0)

        def zacc(i, _):
            pltpu.sync_copy(rows0, acc.at[pl.ds(sid * per_sub + i * b, b)])
            return 0

        lax.fori_loop(0, per_sub // b, zacc, 0)
        zrem = per_sub % b
        if zrem:
            pltpu.sync_copy(
                rows0.at[pl.ds(0, zrem)],
                acc.at[pl.ds(sid * per_sub + (per_sub // b) * b, zrem)])
        plsc.subcore_barrier()

        def load_r(j, buf, sem):
            pltpu.async_copy(r_hbm.at[pl.ds(base + j * b, b)], buf, sem)

        def load_c(j, buf, sem):
            pltpu.async_copy(c_hbm.at[pl.ds(base + j * b, b)], buf, sem)

        def wait(hbm, buf, sem):
            pltpu.make_async_copy(hbm.at[pl.ds(base, b)], buf, sem).wait()

        def gather(buf, rows, sem):
            pltpu.async_copy(y_hbm.at[buf], rows, sem)

        def wait_gather(buf, rows, sem):
            pltpu.make_async_copy(y_hbm.at[buf], rows, sem).wait()

        def scatter(rows, buf):
            pltpu.sync_copy(rows, acc.at[buf], add=True)

        load_r(0, rb0, sir0)
        load_c(0, cb0, sic0)
        load_r(1, rb1, sir1)
        load_c(1, cb1, sic1)
        wait(r_hbm, rb0, sir0)
        gather(rb0, rows0, sg0)
        wait(r_hbm, rb1, sir1)
        gather(rb1, rows1, sg1)

        def pair(i, _):
            j = 2 * i
            wait_gather(rb0, rows0, sg0)
            load_r(j + 2, rb0, sir0)
            wait(c_hbm, cb0, sic0)
            scatter(rows0, cb0)
            load_c(j + 2, cb0, sic0)
            wait(r_hbm, rb0, sir0)
            gather(rb0, rows0, sg0)
            wait_gather(rb1, rows1, sg1)
            load_r(j + 3, rb1, sir1)
            wait(c_hbm, cb1, sic1)
            scatter(rows1, cb1)
            load_c(j + 3, cb1, sic1)
            wait(r_hbm, rb1, sir1)
            gather(rb1, rows1, sg1)
            return 0

        lax.fori_loop(0, ch // 2 - 1, pair, 0)
        wait_gather(rb0, rows0, sg0)
        wait(c_hbm, cb0, sic0)
        scatter(rows0, cb0)
        wait_gather(rb1, rows1, sg1)
        wait(c_hbm, cb1, sic1)
        scatter(rows1, cb1)
        plsc.subcore_barrier()
        pltpu.sync_copy(acc.at[pl.ds(sid * per_sub, per_sub)],
                        out_hbm.at[cid, pl.ds(sid * per_sub, per_sub)])

    return agg_kernel(y, rflat, cflat)


def _tc_pre(x, degp3, w0t, fcwt, fcb2, blk=1000):
    n, d = x.shape
    g = n // blk

    def body(x_ref, degp_ref, w0t_ref, fcwt_ref, fcb_ref,
             y0_ref, res_ref, dinv_ref):
        deg = degp_ref[0] + degp_ref[1] + 1.0
        dinv = lax.rsqrt(deg)
        dinv_ref[...] = dinv
        xb = x_ref[...]
        y0_ref[...] = jnp.dot(xb, w0t_ref[...],
                              preferred_element_type=jnp.float32) * dinv
        res_ref[...] = jnp.dot(xb, fcwt_ref[...],
                               preferred_element_type=jnp.float32) + fcb_ref[...]

    return pl.pallas_call(
        body,
        grid=(g,),
        in_specs=[
            pl.BlockSpec((blk, d), lambda i: (i, 0)),
            pl.BlockSpec((2, blk, 1), lambda i: (0, i, 0)),
            pl.BlockSpec((d, d), lambda i: (0, 0)),
            pl.BlockSpec((d, d), lambda i: (0, 0)),
            pl.BlockSpec((1, d), lambda i: (0, 0)),
        ],
        out_specs=[
            pl.BlockSpec((blk, d), lambda i: (i, 0)),
            pl.BlockSpec((blk, d), lambda i: (i, 0)),
            pl.BlockSpec((blk, 1), lambda i: (i, 0)),
        ],
        out_shape=[
            jax.ShapeDtypeStruct((n, d), jnp.float32),
            jax.ShapeDtypeStruct((n, d), jnp.float32),
            jax.ShapeDtypeStruct((n, 1), jnp.float32),
        ],
    )(x, degp3, w0t, fcwt, fcb2)


def _tc_mid(aggp, y0, dinv, b02, w1t, blk=1000):
    n, d = y0.shape

    def body(aggp_ref, y0_ref, dinv_ref, b0_ref, w1t_ref, y1_ref):
        dinv = dinv_ref[...]
        h = (aggp_ref[0] + aggp_ref[1] + y0_ref[...]) * dinv + b0_ref[...]
        h = jnp.maximum(h, 0.0)
        y1_ref[...] = jnp.dot(h, w1t_ref[...],
                              preferred_element_type=jnp.float32) * dinv

    return pl.pallas_call(
        body,
        grid=(n // blk,),
        in_specs=[
            pl.BlockSpec((2, blk, d), lambda i: (0, i, 0)),
            pl.BlockSpec((blk, d), lambda i: (i, 0)),
            pl.BlockSpec((blk, 1), lambda i: (i, 0)),
            pl.BlockSpec((1, d), lambda i: (0, 0)),
            pl.BlockSpec((d, d), lambda i: (0, 0)),
        ],
        out_specs=pl.BlockSpec((blk, d), lambda i: (i, 0)),
        out_shape=jax.ShapeDtypeStruct((n, d), jnp.float32),
    )(aggp, y0, dinv, b02, w1t)


def _tc_post(aggp, y1, dinv, b12, res, blk=1000):
    n, d = y1.shape

    def body(aggp_ref, y1_ref, dinv_ref, b1_ref, res_ref, out_ref):
        h = (aggp_ref[0] + aggp_ref[1] + y1_ref[...]) * dinv_ref[...] \
            + b1_ref[...]
        out_ref[...] = jnp.maximum(h, 0.0) + res_ref[...]

    return pl.pallas_call(
        body,
        grid=(n // blk,),
        in_specs=[
            pl.BlockSpec((2, blk, d), lambda i: (0, i, 0)),
            pl.BlockSpec((blk, d), lambda i: (i, 0)),
            pl.BlockSpec((blk, 1), lambda i: (i, 0)),
            pl.BlockSpec((1, d), lambda i: (0, 0)),
            pl.BlockSpec((blk, d), lambda i: (i, 0)),
        ],
        out_specs=pl.BlockSpec((blk, d), lambda i: (i, 0)),
        out_shape=jax.ShapeDtypeStruct((n, d), jnp.float32),
    )(aggp, y1, dinv, b12, res)


def kernel(x, edge_index, W0, b0, W1, b1, fcW, fcb):
    n, d = x.shape
    e = edge_index.shape[1]
    ch = -(-e // (_NW * _B))
    ch += ch % 2
    ep = _NW * ch * _B
    r = edge_index[0].astype(jnp.int32)
    c = edge_index[1].astype(jnp.int32)
    pad = ep - e
    rflat = jnp.concatenate([r, jnp.zeros((pad,), jnp.int32)])
    cflat = jnp.concatenate([c, jnp.full((pad,), n, jnp.int32)])
    c3 = cflat.reshape(_NW, ch, _B)

    degp = _sc_degree(c3, n)
    degp3 = degp[:, :n].reshape(2, n, 1)
    y0, res, dinv = _tc_pre(x, degp3, W0.T, fcW.T, fcb.reshape(1, d))
    agg0 = _sc_aggregate(y0, rflat, cflat, ch)[:, :n]
    y1 = _tc_mid(agg0, y0, dinv, b0.reshape(1, d), W1.T)
    agg1 = _sc_aggregate(y1, rflat, cflat, ch)[:, :n]
    return _tc_post(agg1, y1, dinv, b1.reshape(1, d), res)


# speedup vs baseline: 1.5468x; 1.3658x over previous
import functools

import jax
import jax.numpy as jnp
from jax import lax
from jax.experimental import pallas as pl
from jax.experimental.pallas import tpu as pltpu
from jax.experimental.pallas import tpu_sc as plsc

_NW = 32
_B = 128
_LANES = 16


def _pad_shape(n):
    npad = -(-(n + 1) // (_LANES * 8)) * _LANES * 8
    npad = -(-npad // 128) * 128
    return npad, npad // _LANES


def _sc_mesh():
    return plsc.VectorSubcoreMesh(core_axis_name="c", subcore_axis_name="s")


def _sc_degree(c3, n):
    nw, ch, b = c3.shape
    npad, per_sub = _pad_shape(n)

    @functools.partial(
        pl.kernel,
        mesh=_sc_mesh(),
        out_type=jax.ShapeDtypeStruct((2, npad), jnp.float32),
        scratch_types=[
            pltpu.VMEM((ch, b), jnp.int32),
            pltpu.VMEM((b,), jnp.float32),
            pltpu.VMEM_SHARED((npad,), jnp.float32),
        ],
    )
    def deg_kernel(c_hbm, deg_hbm, cidx, ones, acc):
        cid = lax.axis_index("c")
        sid = lax.axis_index("s")
        wid = sid * 2 + cid

        def fill(val):
            def step(i, _):
                ones[pl.ds(i * _LANES, _LANES)] = jnp.full(
                    (_LANES,), val, jnp.float32)
                return 0
            lax.fori_loop(0, b // _LANES, step, 0)

        fill(0.0)

        def zero(i, _):
            pltpu.sync_copy(ones, acc.at[pl.ds(sid * per_sub + i * b, b)])
            return 0

        lax.fori_loop(0, per_sub // b, zero, 0)
        rem = per_sub % b
        if rem:
            pltpu.sync_copy(
                ones.at[pl.ds(0, rem)],
                acc.at[pl.ds(sid * per_sub + (per_sub // b) * b, rem)])
        fill(1.0)
        plsc.subcore_barrier()
        pltpu.sync_copy(c_hbm.at[wid], cidx)

        def chunk(j, _):
            pltpu.sync_copy(ones, acc.at[cidx.at[j]], add=True)
            return 0

        lax.fori_loop(0, ch, chunk, 0)
        plsc.subcore_barrier()

        @pl.when(sid == 0)
        def _():
            pltpu.sync_copy(acc, deg_hbm.at[cid])

    return deg_kernel(c3)


def _sc_aggregate(y, r3, c3, ch):
    n, d = y.shape
    b = _B
    npad, per_sub = _pad_shape(n)

    @functools.partial(
        pl.kernel,
        mesh=_sc_mesh(),
        out_type=jax.ShapeDtypeStruct((2, npad, d), jnp.float32),
        scratch_types=[
            pltpu.VMEM((ch, b), jnp.int32),
            pltpu.VMEM((ch, b), jnp.int32),
            pltpu.VMEM((b, d), jnp.float32),
            pltpu.VMEM_SHARED((npad, d), jnp.float32),
        ],
    )
    def agg_kernel(y_hbm, r_hbm, c_hbm, out_hbm, ridx, cidx, rows, acc):
        cid = lax.axis_index("c")
        sid = lax.axis_index("s")
        wid = sid * 2 + cid

        def zrow(i, _):
            for k in range(d // _LANES):
                rows[i, pl.ds(k * _LANES, _LANES)] = jnp.zeros(
                    (_LANES,), jnp.float32)
            return 0

        lax.fori_loop(0, b, zrow, 0)

        def zacc(i, _):
            pltpu.sync_copy(rows, acc.at[pl.ds(sid * per_sub + i * b, b)])
            return 0

        lax.fori_loop(0, per_sub // b, zacc, 0)
        zrem = per_sub % b
        if zrem:
            pltpu.sync_copy(
                rows.at[pl.ds(0, zrem)],
                acc.at[pl.ds(sid * per_sub + (per_sub // b) * b, zrem)])
        pltpu.sync_copy(r_hbm.at[wid], ridx)
        pltpu.sync_copy(c_hbm.at[wid], cidx)
        plsc.subcore_barrier()

        def chunk(j, _):
            pltpu.sync_copy(y_hbm.at[ridx.at[j]], rows)
            pltpu.sync_copy(rows, acc.at[cidx.at[j]], add=True)
            return 0

        lax.fori_loop(0, ch, chunk, 0)
        plsc.subcore_barrier()
        pltpu.sync_copy(acc.at[pl.ds(sid * per_sub, per_sub)],
                        out_hbm.at[cid, pl.ds(sid * per_sub, per_sub)])

    return agg_kernel(y, r3, c3)


def _tc_pre(x, degp3, w0t, fcwt, fcb2, blk=1000):
    n, d = x.shape
    g = n // blk

    def body(x_ref, degp_ref, w0t_ref, fcwt_ref, fcb_ref,
             y0_ref, res_ref, dinv_ref):
        deg = degp_ref[0] + degp_ref[1] + 1.0
        dinv = lax.rsqrt(deg)
        dinv_ref[...] = dinv
        xb = x_ref[...]
        y0_ref[...] = jnp.dot(xb, w0t_ref[...],
                              preferred_element_type=jnp.float32) * dinv
        res_ref[...] = jnp.dot(xb, fcwt_ref[...],
                               preferred_element_type=jnp.float32) + fcb_ref[...]

    return pl.pallas_call(
        body,
        grid=(g,),
        in_specs=[
            pl.BlockSpec((blk, d), lambda i: (i, 0)),
            pl.BlockSpec((2, blk, 1), lambda i: (0, i, 0)),
            pl.BlockSpec((d, d), lambda i: (0, 0)),
            pl.BlockSpec((d, d), lambda i: (0, 0)),
            pl.BlockSpec((1, d), lambda i: (0, 0)),
        ],
        out_specs=[
            pl.BlockSpec((blk, d), lambda i: (i, 0)),
            pl.BlockSpec((blk, d), lambda i: (i, 0)),
            pl.BlockSpec((blk, 1), lambda i: (i, 0)),
        ],
        out_shape=[
            jax.ShapeDtypeStruct((n, d), jnp.float32),
            jax.ShapeDtypeStruct((n, d), jnp.float32),
            jax.ShapeDtypeStruct((n, 1), jnp.float32),
        ],
    )(x, degp3, w0t, fcwt, fcb2)


def _tc_mid(aggp, y0, dinv, b02, w1t, blk=1000):
    n, d = y0.shape

    def body(aggp_ref, y0_ref, dinv_ref, b0_ref, w1t_ref, y1_ref):
        dinv = dinv_ref[...]
        h = (aggp_ref[0] + aggp_ref[1] + y0_ref[...]) * dinv + b0_ref[...]
        h = jnp.maximum(h, 0.0)
        y1_ref[...] = jnp.dot(h, w1t_ref[...],
                              preferred_element_type=jnp.float32) * dinv

    return pl.pallas_call(
        body,
        grid=(n // blk,),
        in_specs=[
            pl.BlockSpec((2, blk, d), lambda i: (0, i, 0)),
            pl.BlockSpec((blk, d), lambda i: (i, 0)),
            pl.BlockSpec((blk, 1), lambda i: (i, 0)),
            pl.BlockSpec((1, d), lambda i: (0, 0)),
            pl.BlockSpec((d, d), lambda i: (0, 0)),
        ],
        out_specs=pl.BlockSpec((blk, d), lambda i: (i, 0)),
        out_shape=jax.ShapeDtypeStruct((n, d), jnp.float32),
    )(aggp, y0, dinv, b02, w1t)


def _tc_post(aggp, y1, dinv, b12, res, blk=1000):
    n, d = y1.shape

    def body(aggp_ref, y1_ref, dinv_ref, b1_ref, res_ref, out_ref):
        h = (aggp_ref[0] + aggp_ref[1] + y1_ref[...]) * dinv_ref[...] \
            + b1_ref[...]
        out_ref[...] = jnp.maximum(h, 0.0) + res_ref[...]

    return pl.pallas_call(
        body,
        grid=(n // blk,),
        in_specs=[
            pl.BlockSpec((2, blk, d), lambda i: (0, i, 0)),
            pl.BlockSpec((blk, d), lambda i: (i, 0)),
            pl.BlockSpec((blk, 1), lambda i: (i, 0)),
            pl.BlockSpec((1, d), lambda i: (0, 0)),
            pl.BlockSpec((blk, d), lambda i: (i, 0)),
        ],
        out_specs=pl.BlockSpec((blk, d), lambda i: (i, 0)),
        out_shape=jax.ShapeDtypeStruct((n, d), jnp.float32),
    )(aggp, y1, dinv, b12, res)


def kernel(x, edge_index, W0, b0, W1, b1, fcW, fcb):
    n, d = x.shape
    e = edge_index.shape[1]
    ch = -(-e // (_NW * _B))
    ep = _NW * ch * _B
    r = edge_index[0].astype(jnp.int32)
    c = edge_index[1].astype(jnp.int32)
    pad = ep - e
    rflat = jnp.concatenate([r, jnp.zeros((pad,), jnp.int32)])
    cflat = jnp.concatenate([c, jnp.full((pad,), n, jnp.int32)])
    r3 = rflat.reshape(_NW, ch, _B)
    c3 = cflat.reshape(_NW, ch, _B)

    degp = _sc_degree(c3, n)
    degp3 = degp[:, :n].reshape(2, n, 1)
    y0, res, dinv = _tc_pre(x, degp3, W0.T, fcW.T, fcb.reshape(1, d))
    agg0 = _sc_aggregate(y0, r3, c3, ch)[:, :n]
    y1 = _tc_mid(agg0, y0, dinv, b0.reshape(1, d), W1.T)
    agg1 = _sc_aggregate(y1, r3, c3, ch)[:, :n]
    return _tc_post(agg1, y1, dinv, b1.reshape(1, d), res)


# speedup vs baseline: 1.5888x; 1.0272x over previous
import functools

import jax
import jax.numpy as jnp
from jax import lax
from jax.experimental import pallas as pl
from jax.experimental.pallas import tpu as pltpu
from jax.experimental.pallas import tpu_sc as plsc

_NW = 32
_B = 128
_LANES = 16


def _pad_shape(n):
    npad = -(-(n + 1) // (_LANES * 8)) * _LANES * 8
    npad = -(-npad // 128) * 128
    return npad, npad // _LANES


def _sc_mesh():
    return plsc.VectorSubcoreMesh(core_axis_name="c", subcore_axis_name="s")


def _sc_degree(c3, n):
    nw, ch, b = c3.shape
    npad, per_sub = _pad_shape(n)

    @functools.partial(
        pl.kernel,
        mesh=_sc_mesh(),
        out_type=jax.ShapeDtypeStruct((2, npad), jnp.float32),
        scratch_types=[
            pltpu.VMEM((ch, b), jnp.int32),
            pltpu.VMEM((b,), jnp.float32),
            pltpu.VMEM_SHARED((npad,), jnp.float32),
        ],
    )
    def deg_kernel(c_hbm, deg_hbm, cidx, ones, acc):
        cid = lax.axis_index("c")
        sid = lax.axis_index("s")
        wid = sid * 2 + cid

        def fill(val):
            def step(i, _):
                ones[pl.ds(i * _LANES, _LANES)] = jnp.full(
                    (_LANES,), val, jnp.float32)
                return 0
            lax.fori_loop(0, b // _LANES, step, 0)

        fill(0.0)

        def zero(i, _):
            pltpu.sync_copy(ones, acc.at[pl.ds(sid * per_sub + i * b, b)])
            return 0

        lax.fori_loop(0, per_sub // b, zero, 0)
        rem = per_sub % b
        if rem:
            pltpu.sync_copy(
                ones.at[pl.ds(0, rem)],
                acc.at[pl.ds(sid * per_sub + (per_sub // b) * b, rem)])
        fill(1.0)
        plsc.subcore_barrier()
        pltpu.sync_copy(c_hbm.at[wid], cidx)

        def chunk(j, _):
            pltpu.sync_copy(ones, acc.at[cidx.at[j]], add=True)
            return 0

        lax.fori_loop(0, ch, chunk, 0)
        plsc.subcore_barrier()

        @pl.when(sid == 0)
        def _():
            pltpu.sync_copy(acc, deg_hbm.at[cid])

    return deg_kernel(c3)


def _sc_aggregate(y, r3, c3, ch):
    n, d = y.shape
    b = _B
    npad, per_sub = _pad_shape(n)

    @functools.partial(
        pl.kernel,
        mesh=_sc_mesh(),
        out_type=jax.ShapeDtypeStruct((2, npad, d), jnp.float32),
        scratch_types=[
            pltpu.VMEM((ch, b), jnp.int32),
            pltpu.VMEM((ch, b), jnp.int32),
            pltpu.VMEM((b, d), jnp.float32),
            pltpu.VMEM_SHARED((npad, d), jnp.float32),
        ],
    )
    def agg_kernel(y_hbm, r_hbm, c_hbm, out_hbm, ridx, cidx, rows, acc):
        cid = lax.axis_index("c")
        sid = lax.axis_index("s")
        wid = sid * 2 + cid

        def zrow(i, _):
            for k in range(d // _LANES):
                rows[i, pl.ds(k * _LANES, _LANES)] = jnp.zeros(
                    (_LANES,), jnp.float32)
            return 0

        lax.fori_loop(0, b, zrow, 0)

        def zacc(i, _):
            pltpu.sync_copy(rows, acc.at[pl.ds(sid * per_sub + i * b, b)])
            return 0

        lax.fori_loop(0, per_sub // b, zacc, 0)
        zrem = per_sub % b
        if zrem:
            pltpu.sync_copy(
                rows.at[pl.ds(0, zrem)],
                acc.at[pl.ds(sid * per_sub + (per_sub // b) * b, zrem)])
        pltpu.sync_copy(r_hbm.at[wid], ridx)
        pltpu.sync_copy(c_hbm.at[wid], cidx)
        plsc.subcore_barrier()

        def chunk(q, _):
            pltpu.sync_copy(y_hbm.at[ridx.at[q]], rows)
            pltpu.sync_copy(rows, acc.at[cidx.at[q]], add=True)
            return 0

        lax.fori_loop(0, ch, chunk, 0)
        plsc.subcore_barrier()
        pltpu.sync_copy(acc.at[pl.ds(sid * per_sub, per_sub)],
                        out_hbm.at[cid, pl.ds(sid * per_sub, per_sub)])

    return agg_kernel(y, r3, c3)


def _tc_pre(x, degp3, w0t, fcwt, fcb2, blk=1000):
    n, d = x.shape
    g = n // blk

    def body(x_ref, degp_ref, w0t_ref, fcwt_ref, fcb_ref,
             y0_ref, res_ref, dinv_ref):
        deg = degp_ref[0] + degp_ref[1] + 1.0
        dinv = lax.rsqrt(deg)
        dinv_ref[...] = dinv
        xb = x_ref[...]
        y0_ref[...] = jnp.dot(xb, w0t_ref[...],
                              preferred_element_type=jnp.float32) * dinv
        res_ref[...] = jnp.dot(xb, fcwt_ref[...],
                               preferred_element_type=jnp.float32) + fcb_ref[...]

    return pl.pallas_call(
        body,
        grid=(g,),
        in_specs=[
            pl.BlockSpec((blk, d), lambda i: (i, 0)),
            pl.BlockSpec((2, blk, 1), lambda i: (0, i, 0)),
            pl.BlockSpec((d, d), lambda i: (0, 0)),
            pl.BlockSpec((d, d), lambda i: (0, 0)),
            pl.BlockSpec((1, d), lambda i: (0, 0)),
        ],
        out_specs=[
            pl.BlockSpec((blk, d), lambda i: (i, 0)),
            pl.BlockSpec((blk, d), lambda i: (i, 0)),
            pl.BlockSpec((blk, 1), lambda i: (i, 0)),
        ],
        out_shape=[
            jax.ShapeDtypeStruct((n, d), jnp.float32),
            jax.ShapeDtypeStruct((n, d), jnp.float32),
            jax.ShapeDtypeStruct((n, 1), jnp.float32),
        ],
    )(x, degp3, w0t, fcwt, fcb2)


def _tc_mid(aggp, y0, dinv, b02, w1t, blk=1000):
    n, d = y0.shape

    def body(aggp_ref, y0_ref, dinv_ref, b0_ref, w1t_ref, y1_ref):
        dinv = dinv_ref[...]
        h = (aggp_ref[0] + aggp_ref[1] + y0_ref[...]) * dinv + b0_ref[...]
        h = jnp.maximum(h, 0.0)
        y1_ref[...] = jnp.dot(h, w1t_ref[...],
                              preferred_element_type=jnp.float32) * dinv

    return pl.pallas_call(
        body,
        grid=(n // blk,),
        in_specs=[
            pl.BlockSpec((2, blk, d), lambda i: (0, i, 0)),
            pl.BlockSpec((blk, d), lambda i: (i, 0)),
            pl.BlockSpec((blk, 1), lambda i: (i, 0)),
            pl.BlockSpec((1, d), lambda i: (0, 0)),
            pl.BlockSpec((d, d), lambda i: (0, 0)),
        ],
        out_specs=pl.BlockSpec((blk, d), lambda i: (i, 0)),
        out_shape=jax.ShapeDtypeStruct((n, d), jnp.float32),
    )(aggp, y0, dinv, b02, w1t)


def _tc_post(aggp, y1, dinv, b12, res, blk=1000):
    n, d = y1.shape

    def body(aggp_ref, y1_ref, dinv_ref, b1_ref, res_ref, out_ref):
        h = (aggp_ref[0] + aggp_ref[1] + y1_ref[...]) * dinv_ref[...] \
            + b1_ref[...]
        out_ref[...] = jnp.maximum(h, 0.0) + res_ref[...]

    return pl.pallas_call(
        body,
        grid=(n // blk,),
        in_specs=[
            pl.BlockSpec((2, blk, d), lambda i: (0, i, 0)),
            pl.BlockSpec((blk, d), lambda i: (i, 0)),
            pl.BlockSpec((blk, 1), lambda i: (i, 0)),
            pl.BlockSpec((1, d), lambda i: (0, 0)),
            pl.BlockSpec((blk, d), lambda i: (i, 0)),
        ],
        out_specs=pl.BlockSpec((blk, d), lambda i: (i, 0)),
        out_shape=jax.ShapeDtypeStruct((n, d), jnp.float32),
    )(aggp, y1, dinv, b12, res)


def kernel(x, edge_index, W0, b0, W1, b1, fcW, fcb):
    n, d = x.shape
    e = edge_index.shape[1]
    ch = -(-e // (_NW * _B))
    ep = _NW * ch * _B
    r = edge_index[0].astype(jnp.int32)
    c = edge_index[1].astype(jnp.int32)
    pad = ep - e
    rflat = jnp.concatenate([r, jnp.zeros((pad,), jnp.int32)])
    cflat = jnp.concatenate([c, jnp.full((pad,), n, jnp.int32)])
    r3 = rflat.reshape(_NW, ch, _B)
    c3 = cflat.reshape(_NW, ch, _B)

    degp = _sc_degree(c3, n)
    degp3 = degp[:, :n].reshape(2, n, 1)
    y0, res, dinv = _tc_pre(x, degp3, W0.T, fcW.T, fcb.reshape(1, d))
    agg0 = _sc_aggregate(y0, r3, c3, ch)[:, :n]
    y1 = _tc_mid(agg0, y0, dinv, b0.reshape(1, d), W1.T)
    agg1 = _sc_aggregate(y1, r3, c3, ch)[:, :n]
    return _tc_post(agg1, y1, dinv, b1.reshape(1, d), res)
